# baseline (speedup 1.0000x reference)
import jax
import jax.numpy as jnp
from jax import lax
from jax.experimental import pallas as pl
from jax.experimental.pallas import tpu as pltpu
from jax.experimental.pallas import tpu_sc as plsc

NN = 10000
NP = 10240
FD = 128
EE = 320000
NC = 2
NS = 16
NT = NC * NS
EPT = 10240
CW = 128
NCHUNK = EPT // CW
FQ = 32
MCHUNK = (NT * EPT) // NS // CW
RPT = NP // NS
JUNK = NN
BLK = 512
NBLK = NP // BLK
SKIPW = 0.5
EPSB = 1e-5


def _sc_mesh():
    return plsc.VectorSubcoreMesh(
        core_axis_name="c", subcore_axis_name="s",
        num_cores=NC, num_subcores=NS)


def _deg_body(dst_hbm, ones_hbm, zero_hbm, out_hbm, dstv, onesv, acc, ssem):
    cid = lax.axis_index("c")
    sid = lax.axis_index("s")
    pltpu.sync_copy(dst_hbm.at[cid, sid], dstv)
    pltpu.sync_copy(ones_hbm, onesv)
    pltpu.sync_copy(zero_hbm, acc.at[pl.ds(sid * RPT, RPT)])
    plsc.subcore_barrier()

    def body(i, carry):
        j0 = 8 * i
        for k in range(8):
            pltpu.async_copy(onesv, acc.at[dstv.at[j0 + k]], ssem, add=True)
        for k in range(8):
            pltpu.make_async_copy(onesv, acc.at[dstv.at[j0 + k]], ssem).wait()
        return carry

    lax.fori_loop(0, NCHUNK // 8, body, 0)
    plsc.subcore_barrier()
    pltpu.sync_copy(acc.at[pl.ds(sid * RPT, RPT)], out_hbm.at[cid, sid])


def _sc_degree(dstp):
    fn = pl.kernel(
        _deg_body,
        out_type=jax.ShapeDtypeStruct((NC, NS, RPT, 16), jnp.float32),
        mesh=_sc_mesh(),
        compiler_params=pltpu.CompilerParams(use_tc_tiling_on_sc=False),
        scratch_types=[
            pltpu.VMEM((NCHUNK, CW), jnp.int32),
            pltpu.VMEM((CW, 16), jnp.float32),
            pltpu.VMEM_SHARED((NP, 16), jnp.float32),
            pltpu.SemaphoreType.DMA,
        ],
    )
    out = fn(dstp, jnp.ones((CW, 16), jnp.float32),
             jnp.zeros((RPT, 16), jnp.float32))
    return out.reshape(NC, NP, 16)


def _msg_body(ht_hbm, src_hbm, dst_hbm, out_hbm,
              srcv, dstv, b0, b1, b2, b3,
              tabs, acc, g0, g1, g2, g3, s0, s1, s2, s3):
    cid = lax.axis_index("c")
    sid = lax.axis_index("s")
    bufs = (b0, b1, b2, b3)
    gsem = (g0, g1, g2, g3)
    ssem = (s0, s1, s2, s3)
    pltpu.sync_copy(src_hbm.at[sid], srcv)
    pltpu.sync_copy(dst_hbm.at[sid], dstv)
    rows = pl.ds(sid * RPT, RPT)

    for r in range(2):
        q = 2 * cid + r
        pltpu.sync_copy(ht_hbm.at[q, rows], tabs.at[rows])
        pltpu.sync_copy(ht_hbm.at[q, rows], acc.at[rows])
        plsc.subcore_barrier()

        for k in range(4):
            pltpu.async_copy(tabs.at[srcv.at[k]], bufs[k], gsem[k])

        def body(i, carry):
            j0 = 4 * i
            for k in range(4):
                pltpu.make_async_copy(
                    tabs.at[srcv.at[j0 + k]], bufs[k], gsem[k]).wait()
                pltpu.async_copy(
                    bufs[k], acc.at[dstv.at[j0 + k]], ssem[k], add=True)

            @pl.when(i < MCHUNK // 4 - 1)
            def _():
                for k in range(4):
                    pltpu.make_async_copy(
                        bufs[k], acc.at[dstv.at[j0 + k]], ssem[k]).wait()
                    pltpu.async_copy(
                        tabs.at[srcv.at[j0 + 4 + k]], bufs[k], gsem[k])
            return carry

        lax.fori_loop(0, MCHUNK // 4, body, 0)
        jl = MCHUNK - 4
        for k in range(4):
            pltpu.make_async_copy(bufs[k], acc.at[dstv.at[jl + k]],
                                  ssem[k]).wait()
        plsc.subcore_barrier()
        pltpu.sync_copy(acc.at[rows], out_hbm.at[cid, r, sid])


def _sc_message(ht, srcm, dstm):
    fn = pl.kernel(
        _msg_body,
        out_type=jax.ShapeDtypeStruct((NC, 2, NS, RPT, FQ), jnp.float32),
        mesh=_sc_mesh(),
        compiler_params=pltpu.CompilerParams(use_tc_tiling_on_sc=False),
        scratch_types=[
            pltpu.VMEM((MCHUNK, CW), jnp.int32),
            pltpu.VMEM((MCHUNK, CW), jnp.int32),
            pltpu.VMEM((CW, FQ), jnp.float32),
            pltpu.VMEM((CW, FQ), jnp.float32),
            pltpu.VMEM((CW, FQ), jnp.float32),
            pltpu.VMEM((CW, FQ), jnp.float32),
            pltpu.VMEM_SHARED((NP, FQ), jnp.float32),
            pltpu.VMEM_SHARED((NP, FQ), jnp.float32),
            pltpu.SemaphoreType.DMA,
            pltpu.SemaphoreType.DMA,
            pltpu.SemaphoreType.DMA,
            pltpu.SemaphoreType.DMA,
            pltpu.SemaphoreType.DMA,
            pltpu.SemaphoreType.DMA,
            pltpu.SemaphoreType.DMA,
            pltpu.SemaphoreType.DMA,
        ],
    )
    out = fn(ht, srcm, dstm)
    return out.reshape(4, NP, FQ)


def _linht_kernel(x_ref, wi_ref, b_ref, wg_ref, p_ref, h_ref, o_ref):
    h0 = (jnp.dot(x_ref[...], wi_ref[...],
                  preferred_element_type=jnp.float32) + b_ref[...])
    h_ref[...] = h0
    p = p_ref[...]
    dinv = lax.rsqrt(p[0, :, 0] + p[1, :, 0] + 1.0)
    hw = jnp.dot(h0, wg_ref[...], preferred_element_type=jnp.float32)
    hs = hw * dinv[:, None]
    for q in range(4):
        o_ref[q] = hs[:, q * FQ:(q + 1) * FQ]


def _aggz_kernel(a_ref, p_ref, b_ref, z_ref, s_ref):
    i = pl.program_id(0)
    p = p_ref[...]
    dinv = lax.rsqrt(p[0, :, 0] + p[1, :, 0] + 1.0)
    a = jnp.concatenate([a_ref[q] for q in range(4)], axis=1)
    z = a * dinv[:, None] + b_ref[...]
    z_ref[...] = z
    rows = lax.broadcasted_iota(jnp.int32, (BLK, 1), 0) + i * BLK
    zm = jnp.where(rows < NN, z, 0.0)

    @pl.when(i == 0)
    def _():
        s_ref[...] = jnp.zeros_like(s_ref)

    s_ref[0:1, :] += jnp.sum(zm, axis=0, keepdims=True)
    s_ref[1:2, :] += jnp.sum(zm * zm, axis=0, keepdims=True)


def _bnmm_kernel(z_ref, s_ref, p_ref, w_ref, g_ref, be_ref, o1_ref, ht_ref):
    s = s_ref[...]
    m = s[0:1, :] * (1.0 / NN)
    v = s[1:2, :] * (1.0 / NN) - m * m
    binv = lax.rsqrt(v + EPSB)
    o1 = jnp.maximum((z_ref[...] - m) * binv * g_ref[...] + be_ref[...], 0.0)
    o1_ref[...] = o1
    p = p_ref[...]
    dinv = lax.rsqrt(p[0, :, 0] + p[1, :, 0] + 1.0)
    o = jnp.dot(o1, w_ref[...], preferred_element_type=jnp.float32)
    hs = o * dinv[:, None]
    for q in range(4):
        ht_ref[q] = hs[:, q * FQ:(q + 1) * FQ]


def _final_kernel(z_ref, s_ref, h0_ref, o1_ref, w_ref, g_ref, be_ref,
                  bo_ref, y_ref):
    s = s_ref[...]
    m = s[0:1, :] * (1.0 / NN)
    v = s[1:2, :] * (1.0 / NN) - m * m
    binv = lax.rsqrt(v + EPSB)
    t = jnp.maximum((z_ref[...] - m) * binv * g_ref[...] + be_ref[...]
                    + SKIPW * h0_ref[...], 0.0)
    y_ref[...] = (jnp.dot(o1_ref[...] + t, w_ref[...],
                          preferred_element_type=jnp.float32) + bo_ref[...])


def _rowspec():
    return pl.BlockSpec((BLK, FD), lambda i: (i, 0))


def _fullspec(r):
    return pl.BlockSpec((r, FD), lambda i: (0, 0))


def _pspec():
    return pl.BlockSpec((NC, BLK, 16), lambda i: (0, i, 0))


def _hspec():
    return pl.BlockSpec((4, BLK, FQ), lambda i: (0, i, 0))


_ROWS_OUT = jax.ShapeDtypeStruct((NP, FD), jnp.float32)
_HALF_OUT = jax.ShapeDtypeStruct((4, NP, FQ), jnp.float32)
_STATS_OUT = jax.ShapeDtypeStruct((8, FD), jnp.float32)


def _tc_linht(x, wi, b, wg, p):
    return pl.pallas_call(
        _linht_kernel, grid=(NBLK,),
        in_specs=[_rowspec(), _fullspec(FD), _fullspec(1), _fullspec(FD),
                  _pspec()],
        out_specs=[_rowspec(), _hspec()],
        out_shape=[_ROWS_OUT, _HALF_OUT],
    )(x, wi, b.reshape(1, FD), wg, p)


def _tc_aggz(a, p, b):
    return pl.pallas_call(
        _aggz_kernel, grid=(NBLK,),
        in_specs=[_hspec(), _pspec(), _fullspec(1)],
        out_specs=[_rowspec(), pl.BlockSpec((8, FD), lambda i: (0, 0))],
        out_shape=[_ROWS_OUT, _STATS_OUT],
    )(a, p, b.reshape(1, FD))


def _tc_bnmm(z, stats, p, w, g, be):
    return pl.pallas_call(
        _bnmm_kernel, grid=(NBLK,),
        in_specs=[_rowspec(), pl.BlockSpec((8, FD), lambda i: (0, 0)),
                  _pspec(), _fullspec(FD), _fullspec(1), _fullspec(1)],
        out_specs=[_rowspec(), _hspec()],
        out_shape=[_ROWS_OUT, _HALF_OUT],
    )(z, stats, p, w, g.reshape(1, FD), be.reshape(1, FD))


def _tc_final(z, stats, h0, o1, w, g, be, bo):
    return pl.pallas_call(
        _final_kernel, grid=(NBLK,),
        in_specs=[_rowspec(), pl.BlockSpec((8, FD), lambda i: (0, 0)),
                  _rowspec(), _rowspec(), _fullspec(FD), _fullspec(1),
                  _fullspec(1), _fullspec(1)],
        out_specs=_rowspec(), out_shape=_ROWS_OUT,
    )(z, stats, h0, o1, w, g.reshape(1, FD), be.reshape(1, FD),
      bo.reshape(1, FD))


def kernel(x, edge_index, W_in, b_in, Wg1, bg1, g1, be1, Wg2, bg2, g2, be2,
           W_out, b_out):
    src = edge_index[0]
    dst = edge_index[1]
    pad = NT * EPT - EE
    srcf = jnp.concatenate([src, jnp.zeros((pad,), jnp.int32)])
    dstf = jnp.concatenate([dst, jnp.full((pad,), JUNK, jnp.int32)])
    dstp = dstf.reshape(NC, NS, NCHUNK, CW)
    srcm = srcf.reshape(NS, MCHUNK, CW)
    dstm = dstf.reshape(NS, MCHUNK, CW)
    xp = jnp.pad(x, ((0, NP - NN), (0, 0)))

    p = _sc_degree(dstp)
    h0, ht1 = _tc_linht(xp, W_in, b_in, Wg1, p)
    a1 = _sc_message(ht1, srcm, dstm)
    z1, s1 = _tc_aggz(a1, p, bg1)
    o1, ht2 = _tc_bnmm(z1, s1, p, Wg2, g1, be1)
    a2 = _sc_message(ht2, srcm, dstm)
    z2, s2 = _tc_aggz(a2, p, bg2)
    y = _tc_final(z2, s2, h0, o1, W_out, g2, be2, b_out)
    return y[:NN]


# speedup vs baseline: 1.1230x; 1.1230x over previous
import jax
import jax.numpy as jnp
from jax import lax
from jax.experimental import pallas as pl
from jax.experimental.pallas import tpu as pltpu
from jax.experimental.pallas import tpu_sc as plsc

NN = 10000
NP = 10240
FD = 128
EE = 320000
NC = 2
NS = 16
NT = NC * NS
EPT = 10240
CW = 128
NCHUNK = EPT // CW
FQ = 32
MCHUNK = (NT * EPT) // NS // CW
RPT = NP // NS
JUNK = NN
BLK = 512
NBLK = NP // BLK
SKIPW = 0.5
EPSB = 1e-5


def _sc_mesh():
    return plsc.VectorSubcoreMesh(
        core_axis_name="c", subcore_axis_name="s",
        num_cores=NC, num_subcores=NS)


def _deg_body(dst_hbm, ones_hbm, zero_hbm, out_hbm, dstv, onesv, acc, ssem):
    cid = lax.axis_index("c")
    sid = lax.axis_index("s")
    pltpu.sync_copy(dst_hbm.at[cid, sid], dstv)
    pltpu.sync_copy(ones_hbm, onesv)
    pltpu.sync_copy(zero_hbm, acc.at[pl.ds(sid * RPT, RPT)])
    plsc.subcore_barrier()

    def body(i, carry):
        j0 = 8 * i
        for k in range(8):
            pltpu.async_copy(onesv, acc.at[dstv.at[j0 + k]], ssem, add=True)
        for k in range(8):
            pltpu.make_async_copy(onesv, acc.at[dstv.at[j0 + k]], ssem).wait()
        return carry

    lax.fori_loop(0, NCHUNK // 8, body, 0)
    plsc.subcore_barrier()
    pltpu.sync_copy(acc.at[pl.ds(sid * RPT, RPT)], out_hbm.at[cid, sid])


def _sc_degree(dstp):
    fn = pl.kernel(
        _deg_body,
        out_type=jax.ShapeDtypeStruct((NC, NS, RPT, 16), jnp.float32),
        mesh=_sc_mesh(),
        compiler_params=pltpu.CompilerParams(use_tc_tiling_on_sc=False),
        scratch_types=[
            pltpu.VMEM((NCHUNK, CW), jnp.int32),
            pltpu.VMEM((CW, 16), jnp.float32),
            pltpu.VMEM_SHARED((NP, 16), jnp.float32),
            pltpu.SemaphoreType.DMA,
        ],
    )
    out = fn(dstp, jnp.ones((CW, 16), jnp.float32),
             jnp.zeros((RPT, 16), jnp.float32))
    return out.reshape(NC, NP, 16)


def _msg_body(ht_hbm, src_hbm, dst_hbm, sidx_hbm, out_hbm,
              srcv, dstv, b0, b1, b2, b3, idxv, sbuf,
              tabs, acc, g0, g1, g2, g3, s0, s1, s2, s3):
    cid = lax.axis_index("c")
    sid = lax.axis_index("s")
    bufs = (b0, b1, b2, b3)
    gsem = (g0, g1, g2, g3)
    ssem = (s0, s1, s2, s3)
    pltpu.sync_copy(src_hbm.at[sid], srcv)
    pltpu.sync_copy(dst_hbm.at[sid], dstv)

    for r in range(2):
        q = 2 * cid + r
        pltpu.sync_copy(sidx_hbm.at[q, sid], idxv)
        for b in range(RPT // CW):
            pltpu.async_copy(ht_hbm.at[idxv.at[b]], sbuf, g0).wait()
            seg = pl.ds(sid * RPT + b * CW, CW)
            pltpu.sync_copy(sbuf, tabs.at[seg])
            pltpu.sync_copy(sbuf, acc.at[seg])
        plsc.subcore_barrier()

        for k in range(4):
            pltpu.async_copy(tabs.at[srcv.at[k]], bufs[k], gsem[k])

        def body(i, carry):
            j0 = 4 * i
            for k in range(4):
                pltpu.make_async_copy(
                    tabs.at[srcv.at[j0 + k]], bufs[k], gsem[k]).wait()
                pltpu.async_copy(
                    bufs[k], acc.at[dstv.at[j0 + k]], ssem[k], add=True)

            @pl.when(i < MCHUNK // 4 - 1)
            def _():
                for k in range(4):
                    pltpu.make_async_copy(
                        bufs[k], acc.at[dstv.at[j0 + k]], ssem[k]).wait()
                    pltpu.async_copy(
                        tabs.at[srcv.at[j0 + 4 + k]], bufs[k], gsem[k])
            return carry

        lax.fori_loop(0, MCHUNK // 4, body, 0)
        jl = MCHUNK - 4
        for k in range(4):
            pltpu.make_async_copy(bufs[k], acc.at[dstv.at[jl + k]],
                                  ssem[k]).wait()
        plsc.subcore_barrier()
        for b in range(RPT // CW):
            seg = pl.ds(sid * RPT + b * CW, CW)
            pltpu.sync_copy(acc.at[seg], sbuf)
            pltpu.async_copy(sbuf, out_hbm.at[idxv.at[b]], s0).wait()


def _sc_message(ht, srcm, dstm, sidx):
    fn = pl.kernel(
        _msg_body,
        out_type=jax.ShapeDtypeStruct((4 * NP, FQ), jnp.float32),
        mesh=_sc_mesh(),
        compiler_params=pltpu.CompilerParams(use_tc_tiling_on_sc=False),
        scratch_types=[
            pltpu.VMEM((MCHUNK, CW), jnp.int32),
            pltpu.VMEM((MCHUNK, CW), jnp.int32),
            pltpu.VMEM((CW, FQ), jnp.float32),
            pltpu.VMEM((CW, FQ), jnp.float32),
            pltpu.VMEM((CW, FQ), jnp.float32),
            pltpu.VMEM((CW, FQ), jnp.float32),
            pltpu.VMEM((RPT // CW, CW), jnp.int32),
            pltpu.VMEM((CW, FQ), jnp.float32),
            pltpu.VMEM_SHARED((NP, FQ), jnp.float32),
            pltpu.VMEM_SHARED((NP, FQ), jnp.float32),
            pltpu.SemaphoreType.DMA,
            pltpu.SemaphoreType.DMA,
            pltpu.SemaphoreType.DMA,
            pltpu.SemaphoreType.DMA,
            pltpu.SemaphoreType.DMA,
            pltpu.SemaphoreType.DMA,
            pltpu.SemaphoreType.DMA,
            pltpu.SemaphoreType.DMA,
        ],
    )
    out = fn(ht.reshape(4 * NP, FQ), srcm, dstm, sidx)
    return out.reshape(NP, FD)


def _linht_kernel(x_ref, wi_ref, b_ref, wg_ref, p_ref, h_ref, o_ref):
    h0 = (jnp.dot(x_ref[...], wi_ref[...],
                  preferred_element_type=jnp.float32) + b_ref[...])
    h_ref[...] = h0
    p = p_ref[...]
    dinv = lax.rsqrt(p[0, :, 0] + p[1, :, 0] + 1.0)
    hw = jnp.dot(h0, wg_ref[...], preferred_element_type=jnp.float32)
    o_ref[...] = hw * dinv[:, None]


def _aggz_kernel(a_ref, p_ref, b_ref, z_ref, s_ref):
    i = pl.program_id(0)
    p = p_ref[...]
    dinv = lax.rsqrt(p[0, :, 0] + p[1, :, 0] + 1.0)
    z = a_ref[...] * dinv[:, None] + b_ref[...]
    z_ref[...] = z
    rows = lax.broadcasted_iota(jnp.int32, (BLK, 1), 0) + i * BLK
    zm = jnp.where(rows < NN, z, 0.0)

    @pl.when(i == 0)
    def _():
        s_ref[...] = jnp.zeros_like(s_ref)

    s_ref[0:1, :] += jnp.sum(zm, axis=0, keepdims=True)
    s_ref[1:2, :] += jnp.sum(zm * zm, axis=0, keepdims=True)


def _bnmm_kernel(z_ref, s_ref, p_ref, w_ref, g_ref, be_ref, o1_ref, ht_ref):
    s = s_ref[...]
    m = s[0:1, :] * (1.0 / NN)
    v = s[1:2, :] * (1.0 / NN) - m * m
    binv = lax.rsqrt(v + EPSB)
    o1 = jnp.maximum((z_ref[...] - m) * binv * g_ref[...] + be_ref[...], 0.0)
    o1_ref[...] = o1
    p = p_ref[...]
    dinv = lax.rsqrt(p[0, :, 0] + p[1, :, 0] + 1.0)
    o = jnp.dot(o1, w_ref[...], preferred_element_type=jnp.float32)
    ht_ref[...] = o * dinv[:, None]


def _final_kernel(z_ref, s_ref, h0_ref, o1_ref, w_ref, g_ref, be_ref,
                  bo_ref, y_ref):
    s = s_ref[...]
    m = s[0:1, :] * (1.0 / NN)
    v = s[1:2, :] * (1.0 / NN) - m * m
    binv = lax.rsqrt(v + EPSB)
    t = jnp.maximum((z_ref[...] - m) * binv * g_ref[...] + be_ref[...]
                    + SKIPW * h0_ref[...], 0.0)
    y_ref[...] = (jnp.dot(o1_ref[...] + t, w_ref[...],
                          preferred_element_type=jnp.float32) + bo_ref[...])


def _rowspec():
    return pl.BlockSpec((BLK, FD), lambda i: (i, 0))


def _fullspec(r):
    return pl.BlockSpec((r, FD), lambda i: (0, 0))


def _pspec():
    return pl.BlockSpec((NC, BLK, 16), lambda i: (0, i, 0))


_ROWS_OUT = jax.ShapeDtypeStruct((NP, FD), jnp.float32)
_STATS_OUT = jax.ShapeDtypeStruct((8, FD), jnp.float32)


def _tc_linht(x, wi, b, wg, p):
    return pl.pallas_call(
        _linht_kernel, grid=(NBLK,),
        in_specs=[_rowspec(), _fullspec(FD), _fullspec(1), _fullspec(FD),
                  _pspec()],
        out_specs=[_rowspec(), _rowspec()],
        out_shape=[_ROWS_OUT, _ROWS_OUT],
    )(x, wi, b.reshape(1, FD), wg, p)


def _tc_aggz(a, p, b):
    return pl.pallas_call(
        _aggz_kernel, grid=(NBLK,),
        in_specs=[_rowspec(), _pspec(), _fullspec(1)],
        out_specs=[_rowspec(), pl.BlockSpec((8, FD), lambda i: (0, 0))],
        out_shape=[_ROWS_OUT, _STATS_OUT],
    )(a, p, b.reshape(1, FD))


def _tc_bnmm(z, stats, p, w, g, be):
    return pl.pallas_call(
        _bnmm_kernel, grid=(NBLK,),
        in_specs=[_rowspec(), pl.BlockSpec((8, FD), lambda i: (0, 0)),
                  _pspec(), _fullspec(FD), _fullspec(1), _fullspec(1)],
        out_specs=[_rowspec(), _rowspec()],
        out_shape=[_ROWS_OUT, _ROWS_OUT],
    )(z, stats, p, w, g.reshape(1, FD), be.reshape(1, FD))


def _tc_final(z, stats, h0, o1, w, g, be, bo):
    return pl.pallas_call(
        _final_kernel, grid=(NBLK,),
        in_specs=[_rowspec(), pl.BlockSpec((8, FD), lambda i: (0, 0)),
                  _rowspec(), _rowspec(), _fullspec(FD), _fullspec(1),
                  _fullspec(1), _fullspec(1)],
        out_specs=_rowspec(), out_shape=_ROWS_OUT,
    )(z, stats, h0, o1, w, g.reshape(1, FD), be.reshape(1, FD),
      bo.reshape(1, FD))


def kernel(x, edge_index, W_in, b_in, Wg1, bg1, g1, be1, Wg2, bg2, g2, be2,
           W_out, b_out):
    src = edge_index[0]
    dst = edge_index[1]
    pad = NT * EPT - EE
    srcf = jnp.concatenate([src, jnp.zeros((pad,), jnp.int32)])
    dstf = jnp.concatenate([dst, jnp.full((pad,), JUNK, jnp.int32)])
    dstp = dstf.reshape(NC, NS, NCHUNK, CW)
    srcm = srcf.reshape(NS, MCHUNK, CW)
    dstm = dstf.reshape(NS, MCHUNK, CW)
    xp = jnp.pad(x, ((0, NP - NN), (0, 0)))
    sidx = (jnp.arange(NP, dtype=jnp.int32)[None, :] * 4
            + jnp.arange(4, dtype=jnp.int32)[:, None]
            ).reshape(4, NS, RPT // CW, CW)

    p = _sc_degree(dstp)
    h0, ht1 = _tc_linht(xp, W_in, b_in, Wg1, p)
    a1 = _sc_message(ht1, srcm, dstm, sidx)
    z1, s1 = _tc_aggz(a1, p, bg1)
    o1, ht2 = _tc_bnmm(z1, s1, p, Wg2, g1, be1)
    a2 = _sc_message(ht2, srcm, dstm, sidx)
    z2, s2 = _tc_aggz(a2, p, bg2)
    y = _tc_final(z2, s2, h0, o1, W_out, g2, be2, b_out)
    return y[:NN]
